# baseline (speedup 1.0000x reference)
import functools

import jax
import jax.numpy as jnp
import numpy as np
from jax import lax
from jax.experimental import pallas as pl
from jax.experimental.pallas import tpu as pltpu
from jax.sharding import Mesh, PartitionSpec as P

try:
    from jax import shard_map as _shard_map

    def _smap(f, mesh, in_specs, out_specs):
        return _shard_map(f, mesh=mesh, in_specs=in_specs,
                          out_specs=out_specs, check_vma=False)
except ImportError:
    from jax.experimental.shard_map import shard_map as _shard_map

    def _smap(f, mesh, in_specs, out_specs):
        return _shard_map(f, mesh=mesh, in_specs=in_specs,
                          out_specs=out_specs, check_rep=False)

_NC = 4


def _u_chunk_copy(u_hbm, u_vmem, sems, j, cj):
    return pltpu.make_async_copy(
        u_hbm.at[:, pl.ds(j * cj, cj)],
        u_vmem.at[:, pl.ds(j * cj, cj)],
        sems.at[j],
    )


def _partial_kernel(ve_ref, la_ref, x_ref, w1_ref, u_hbm,
                    hp_ref,
                    u_vmem, t2_ref, sems):
    f32 = jnp.float32
    nh = u_vmem.shape[1]
    cj = nh // _NC

    for j in range(_NC):
        _u_chunk_copy(u_hbm, u_vmem, sems, j, cj).start()

    X = x_ref[...]

    for j in range(_NC):
        _u_chunk_copy(u_hbm, u_vmem, sems, j, cj).wait()
        t2_ref[:, pl.ds(j * cj, cj)] = lax.dot_general(
            X, u_vmem[:, pl.ds(j * cj, cj)], (((0,), (0,)), ((), ())),
            preferred_element_type=f32)

    vla = jnp.power(la_ref[...], ve_ref[0])
    Tv2 = t2_ref[...] * vla

    Tw2 = lax.dot_general(w1_ref[...], Tv2, (((1,), (0,)), ((), ())),
                          preferred_element_type=f32)

    hp_ref[...] = lax.dot_general(u_vmem[...], Tw2, (((1,), (1,)), ((), ())),
                                  preferred_element_type=f32)


def _combine_kernel(idx_ref, h_ref, b1_ref, gamma_ref, beta_ref,
                    w2_ref, b2_ref,
                    out_ref, hid_ref):
    f32 = jnp.float32
    nh = hid_ref.shape[0]
    c = idx_ref[0]

    H = h_ref[...] + b1_ref[...]

    mean = jnp.mean(H, axis=0, keepdims=True)
    var = jnp.mean(jnp.square(H - mean), axis=0, keepdims=True)

    Hh = h_ref[pl.ds(c * nh, nh), :] + b1_ref[...]
    hid_ref[...] = Hh

    Hn = (Hh - mean) * lax.rsqrt(var + 1e-5)
    Hn = Hn * gamma_ref[...] + beta_ref[...]
    Hr = jnp.maximum(Hn, 0.0)

    logits = lax.dot_general(Hr, w2_ref[...], (((1,), (1,)), ((), ())),
                             preferred_element_type=f32) + b2_ref[...]

    m = jnp.max(logits, axis=1, keepdims=True)
    z = logits - m
    lse = jnp.log(jnp.sum(jnp.exp(z), axis=1, keepdims=True))
    out_ref[...] = z - lse


def _sharded_fwd(ve, la_c, x, w1, u_c, b1, gamma, beta, w2, b2):
    f32 = jnp.float32
    N, in_c = x.shape
    hidden = w1.shape[0]
    out_c = w2.shape[0]
    nh = N // 2

    vmem = pl.BlockSpec(memory_space=pltpu.MemorySpace.VMEM)
    smem = pl.BlockSpec(memory_space=pltpu.MemorySpace.SMEM)
    hbm = pl.BlockSpec(memory_space=pltpu.MemorySpace.HBM)

    hp = pl.pallas_call(
        _partial_kernel,
        out_shape=jax.ShapeDtypeStruct((N, hidden), f32),
        in_specs=[smem, vmem, vmem, vmem, hbm],
        out_specs=vmem,
        scratch_shapes=[
            pltpu.VMEM((N, nh), f32),
            pltpu.VMEM((in_c, nh), f32),
            pltpu.SemaphoreType.DMA((_NC,)),
        ],
    )(ve, la_c, x, w1, u_c)

    H = lax.psum(hp, "c")

    idx = lax.axis_index("c").astype(jnp.int32).reshape(1)
    out_h, hid_h = pl.pallas_call(
        _combine_kernel,
        out_shape=(
            jax.ShapeDtypeStruct((nh, out_c), f32),
            jax.ShapeDtypeStruct((nh, hidden), f32),
        ),
        in_specs=[smem] + [vmem] * 6,
        out_specs=(vmem, vmem),
    )(idx, H, b1, gamma, beta, w2, b2)
    return out_h, hid_h


def kernel(X, La, U, ve, W_w, W_b, bn_gamma, bn_beta, MLP_w, MLP_b):
    N, in_c = X.shape
    hidden = W_w.shape[0]
    out_c = MLP_w.shape[0]

    f32 = jnp.float32
    devs = jax.devices()[:2]
    mesh = Mesh(np.array(devs), ("c",))

    f = _smap(
        _sharded_fwd, mesh,
        (P(), P(None, "c"), P(), P(), P(None, "c"),
         P(), P(), P(), P(), P()),
        (P("c", None), P("c", None)),
    )
    return f(
        ve.astype(f32).reshape(1),
        La.reshape(1, N).astype(f32),
        X.astype(f32),
        W_w.astype(f32),
        U.astype(f32),
        W_b.reshape(1, hidden).astype(f32),
        bn_gamma.reshape(1, hidden).astype(f32),
        bn_beta.reshape(1, hidden).astype(f32),
        MLP_w.astype(f32),
        MLP_b.reshape(1, out_c).astype(f32),
    )


# speedup vs baseline: 24.6692x; 24.6692x over previous
import functools

import jax
import jax.numpy as jnp
from jax import lax
from jax.experimental import pallas as pl
from jax.experimental.pallas import tpu as pltpu

_NR = 8


def _u_chunk_copy(u_hbm, uf_ref, sems, r, ch):
    return pltpu.make_async_copy(
        u_hbm.at[pl.ds(r * ch, ch), :],
        uf_ref.at[pl.ds(r * ch, ch), :],
        sems.at[r],
    )


def _fused_kernel(ve_ref, la_ref, x_ref,
                  w1_ref, b1_ref, gamma_ref, beta_ref,
                  w2_ref, b2_ref, u_hbm,
                  out_ref, hid_ref,
                  uf_ref, ub_ref, t2_ref, sems):
    f32 = jnp.float32
    bf16 = jnp.bfloat16
    N = u_hbm.shape[0]
    ch = N // _NR

    for r in range(_NR):
        _u_chunk_copy(u_hbm, uf_ref, sems, r, ch).start()

    Xb = x_ref[...].astype(bf16)

    for r in range(_NR):
        _u_chunk_copy(u_hbm, uf_ref, sems, r, ch).wait()
        rows = pl.ds(r * ch, ch)
        ub_ref[rows, :] = uf_ref[rows, :].astype(bf16)
        part = lax.dot_general(
            Xb[r * ch:(r + 1) * ch, :], ub_ref[rows, :],
            (((0,), (0,)), ((), ())), preferred_element_type=f32)
        if r == 0:
            t2_ref[...] = part
        else:
            t2_ref[...] = t2_ref[...] + part

    vla = jnp.power(la_ref[...], ve_ref[0])
    Tv2 = t2_ref[...] * vla

    Tw2 = lax.dot_general(w1_ref[...], Tv2, (((1,), (0,)), ((), ())),
                          preferred_element_type=f32)

    H = lax.dot_general(ub_ref[...], Tw2.astype(bf16),
                        (((1,), (1,)), ((), ())),
                        preferred_element_type=f32) + b1_ref[...]
    hid_ref[...] = H

    mean = jnp.mean(H, axis=0, keepdims=True)
    var = jnp.mean(jnp.square(H - mean), axis=0, keepdims=True)
    Hn = (H - mean) * lax.rsqrt(var + 1e-5)
    Hn = Hn * gamma_ref[...] + beta_ref[...]

    Hr = jnp.maximum(Hn, 0.0)

    logits = lax.dot_general(Hr, w2_ref[...], (((1,), (1,)), ((), ())),
                             preferred_element_type=f32) + b2_ref[...]

    m = jnp.max(logits, axis=1, keepdims=True)
    z = logits - m
    lse = jnp.log(jnp.sum(jnp.exp(z), axis=1, keepdims=True))
    out_ref[...] = z - lse


def kernel(X, La, U, ve, W_w, W_b, bn_gamma, bn_beta, MLP_w, MLP_b):
    N, in_c = X.shape
    hidden = W_w.shape[0]
    out_c = MLP_w.shape[0]

    vmem = pl.BlockSpec(memory_space=pltpu.MemorySpace.VMEM)
    smem = pl.BlockSpec(memory_space=pltpu.MemorySpace.SMEM)
    hbm = pl.BlockSpec(memory_space=pltpu.MemorySpace.HBM)

    out, hidden_emd = pl.pallas_call(
        _fused_kernel,
        out_shape=(
            jax.ShapeDtypeStruct((N, out_c), jnp.float32),
            jax.ShapeDtypeStruct((N, hidden), jnp.float32),
        ),
        in_specs=[smem] + [vmem] * 8 + [hbm],
        out_specs=(vmem, vmem),
        scratch_shapes=[
            pltpu.VMEM((N, N), jnp.float32),
            pltpu.VMEM((N, N), jnp.bfloat16),
            pltpu.VMEM((in_c, N), jnp.float32),
            pltpu.SemaphoreType.DMA((_NR,)),
        ],
    )(
        ve.astype(jnp.float32).reshape(1),
        La.reshape(1, N).astype(jnp.float32),
        X.astype(jnp.float32),
        W_w.astype(jnp.float32),
        W_b.reshape(1, hidden).astype(jnp.float32),
        bn_gamma.reshape(1, hidden).astype(jnp.float32),
        bn_beta.reshape(1, hidden).astype(jnp.float32),
        MLP_w.astype(jnp.float32),
        MLP_b.reshape(1, out_c).astype(jnp.float32),
        U.astype(jnp.float32),
    )
    return out, hidden_emd


# speedup vs baseline: 24.8241x; 1.0063x over previous
import functools

import jax
import jax.numpy as jnp
from jax import lax
from jax.experimental import pallas as pl
from jax.experimental.pallas import tpu as pltpu


def _fused_kernel(ve_ref, la_ref, x_ref,
                  w1_ref, b1_ref, gamma_ref, beta_ref,
                  w2_ref, b2_ref, u_ref,
                  out_ref, hid_ref):
    f32 = jnp.float32
    bf16 = jnp.bfloat16

    Ub = u_ref[...].astype(bf16)
    Xb = x_ref[...].astype(bf16)

    T2 = lax.dot_general(Xb, Ub, (((0,), (0,)), ((), ())),
                         preferred_element_type=f32)

    vla = jnp.power(la_ref[...], ve_ref[0])
    Tv2 = T2 * vla

    Tw2 = lax.dot_general(w1_ref[...], Tv2, (((1,), (0,)), ((), ())),
                          preferred_element_type=f32)

    H = lax.dot_general(Ub, Tw2.astype(bf16), (((1,), (1,)), ((), ())),
                        preferred_element_type=f32) + b1_ref[...]
    hid_ref[...] = H

    mean = jnp.mean(H, axis=0, keepdims=True)
    var = jnp.mean(jnp.square(H - mean), axis=0, keepdims=True)
    Hn = (H - mean) * lax.rsqrt(var + 1e-5)
    Hn = Hn * gamma_ref[...] + beta_ref[...]

    Hr = jnp.maximum(Hn, 0.0)

    logits = lax.dot_general(Hr, w2_ref[...], (((1,), (1,)), ((), ())),
                             preferred_element_type=f32) + b2_ref[...]

    m = jnp.max(logits, axis=1, keepdims=True)
    z = logits - m
    lse = jnp.log(jnp.sum(jnp.exp(z), axis=1, keepdims=True))
    out_ref[...] = z - lse


def kernel(X, La, U, ve, W_w, W_b, bn_gamma, bn_beta, MLP_w, MLP_b):
    N, in_c = X.shape
    hidden = W_w.shape[0]
    out_c = MLP_w.shape[0]

    vmem = pl.BlockSpec(memory_space=pltpu.MemorySpace.VMEM)
    smem = pl.BlockSpec(memory_space=pltpu.MemorySpace.SMEM)

    out, hidden_emd = pl.pallas_call(
        _fused_kernel,
        out_shape=(
            jax.ShapeDtypeStruct((N, out_c), jnp.float32),
            jax.ShapeDtypeStruct((N, hidden), jnp.float32),
        ),
        in_specs=[smem] + [vmem] * 9,
        out_specs=(vmem, vmem),
    )(
        ve.astype(jnp.float32).reshape(1),
        La.reshape(1, N).astype(jnp.float32),
        X.astype(jnp.float32),
        W_w.astype(jnp.float32),
        W_b.reshape(1, hidden).astype(jnp.float32),
        bn_gamma.reshape(1, hidden).astype(jnp.float32),
        bn_beta.reshape(1, hidden).astype(jnp.float32),
        MLP_w.astype(jnp.float32),
        MLP_b.reshape(1, out_c).astype(jnp.float32),
        U.astype(jnp.float32),
    )
    return out, hidden_emd


# speedup vs baseline: 24.8784x; 1.0022x over previous
import jax
import jax.numpy as jnp
from jax import lax
from jax.experimental import pallas as pl
from jax.experimental.pallas import tpu as pltpu


def _fused_kernel(ve_ref, la_ref, x_ref,
                  w1_ref, b1_ref, gamma_ref, beta_ref,
                  w2_ref, b2_ref, u_ref,
                  out_ref, hid_ref):
    f32 = jnp.float32
    bf16 = jnp.bfloat16

    Ub = u_ref[...].astype(bf16)
    Xb = x_ref[...].astype(bf16)

    T2 = lax.dot_general(Xb, Ub, (((0,), (0,)), ((), ())),
                         preferred_element_type=f32)

    vla = jnp.power(la_ref[...], ve_ref[0])
    Tv2 = T2 * vla

    Tw2 = lax.dot_general(w1_ref[...], Tv2, (((1,), (0,)), ((), ())),
                          preferred_element_type=f32)

    H = lax.dot_general(Ub, Tw2.astype(bf16), (((1,), (1,)), ((), ())),
                        preferred_element_type=f32) + b1_ref[...]
    hid_ref[...] = H

    mean = jnp.mean(H, axis=0, keepdims=True)
    var = jnp.mean(jnp.square(H - mean), axis=0, keepdims=True)
    Hn = (H - mean) * lax.rsqrt(var + 1e-5)
    Hn = Hn * gamma_ref[...] + beta_ref[...]

    Hr = jnp.maximum(Hn, 0.0)

    logits = lax.dot_general(Hr, w2_ref[...], (((1,), (1,)), ((), ())),
                             preferred_element_type=f32) + b2_ref[...]

    m = jnp.max(logits, axis=1, keepdims=True)
    z = logits - m
    lse = jnp.log(jnp.sum(jnp.exp(z), axis=1, keepdims=True))
    out_ref[...] = z - lse


def kernel(X, La, U, ve, W_w, W_b, bn_gamma, bn_beta, MLP_w, MLP_b):
    N, in_c = X.shape
    hidden = W_w.shape[0]
    out_c = MLP_w.shape[0]

    vmem = pl.BlockSpec(memory_space=pltpu.MemorySpace.VMEM)
    smem = pl.BlockSpec(memory_space=pltpu.MemorySpace.SMEM)

    out, hidden_emd = pl.pallas_call(
        _fused_kernel,
        out_shape=(
            jax.ShapeDtypeStruct((N, out_c), jnp.float32),
            jax.ShapeDtypeStruct((N, hidden), jnp.float32),
        ),
        in_specs=[smem] + [vmem] * 9,
        out_specs=(vmem, vmem),
    )(
        ve.astype(jnp.float32).reshape(1),
        La.reshape(1, N).astype(jnp.float32),
        X.astype(jnp.float32),
        W_w.astype(jnp.float32),
        W_b.reshape(1, hidden).astype(jnp.float32),
        bn_gamma.reshape(1, hidden).astype(jnp.float32),
        bn_beta.reshape(1, hidden).astype(jnp.float32),
        MLP_w.astype(jnp.float32),
        MLP_b.reshape(1, out_c).astype(jnp.float32),
        U.astype(jnp.float32),
    )
    return out, hidden_emd


# speedup vs baseline: 25.4753x; 1.0240x over previous
import jax
import jax.numpy as jnp
from jax import lax
from jax.experimental import pallas as pl
from jax.experimental.pallas import tpu as pltpu


def _fused_kernel(ve_ref, la_ref, x_ref,
                  w1_ref, b1_ref, gamma_ref, beta_ref,
                  w2_ref, b2_ref, u_ref,
                  out_ref, hid_ref):
    f32 = jnp.float32
    bf16 = jnp.bfloat16

    Ub = u_ref[...].astype(bf16)
    Xb = x_ref[...].astype(bf16)

    T2 = lax.dot_general(Xb, Ub, (((0,), (0,)), ((), ())),
                         preferred_element_type=f32)

    vla = jnp.power(la_ref[...], ve_ref[0])
    Tv2 = T2 * vla

    Tw2 = lax.dot_general(w1_ref[...], Tv2, (((1,), (0,)), ((), ())),
                          preferred_element_type=f32)

    H = lax.dot_general(Ub, Tw2.astype(bf16), (((1,), (1,)), ((), ())),
                        preferred_element_type=f32) + b1_ref[...]
    hid_ref[...] = H

    inv_n = 1.0 / H.shape[0]
    mean = jnp.sum(H, axis=0, keepdims=True) * inv_n
    m2 = jnp.sum(H * H, axis=0, keepdims=True) * inv_n
    var = m2 - mean * mean
    scale = gamma_ref[...] * lax.rsqrt(var + 1e-5)
    shift = beta_ref[...] - mean * scale
    Hr = jnp.maximum(H * scale + shift, 0.0)

    logits = lax.dot_general(Hr, w2_ref[...], (((1,), (1,)), ((), ())),
                             preferred_element_type=f32) + b2_ref[...]

    m = jnp.max(logits, axis=1, keepdims=True)
    z = logits - m
    lse = jnp.log(jnp.sum(jnp.exp(z), axis=1, keepdims=True))
    out_ref[...] = z - lse


def kernel(X, La, U, ve, W_w, W_b, bn_gamma, bn_beta, MLP_w, MLP_b):
    N, in_c = X.shape
    hidden = W_w.shape[0]
    out_c = MLP_w.shape[0]

    vmem = pl.BlockSpec(memory_space=pltpu.MemorySpace.VMEM)
    smem = pl.BlockSpec(memory_space=pltpu.MemorySpace.SMEM)

    out, hidden_emd = pl.pallas_call(
        _fused_kernel,
        out_shape=(
            jax.ShapeDtypeStruct((N, out_c), jnp.float32),
            jax.ShapeDtypeStruct((N, hidden), jnp.float32),
        ),
        in_specs=[smem] + [vmem] * 9,
        out_specs=(vmem, vmem),
    )(
        ve.astype(jnp.float32).reshape(1),
        La.reshape(1, N).astype(jnp.float32),
        X.astype(jnp.float32),
        W_w.astype(jnp.float32),
        W_b.reshape(1, hidden).astype(jnp.float32),
        bn_gamma.reshape(1, hidden).astype(jnp.float32),
        bn_beta.reshape(1, hidden).astype(jnp.float32),
        MLP_w.astype(jnp.float32),
        MLP_b.reshape(1, out_c).astype(jnp.float32),
        U.astype(jnp.float32),
    )
    return out, hidden_emd
